# Initial kernel scaffold; baseline (speedup 1.0000x reference)
import jax
import jax.numpy as jnp
from jax.experimental import pallas as pl


def kernel(t, log_snr):
    raise NotImplementedError("write your pallas kernel here")


# speedup vs baseline: 11.1000x; 11.1000x over previous
import functools

import jax
import jax.numpy as jnp
from jax import lax
from jax.experimental import pallas as pl
from jax.experimental.pallas import tpu as pltpu
from jax.experimental.pallas import tpu_sc as plsc

NC = 2
NS = 16
NW = NC * NS
L = 16

B = 16384
BPW = B // NW
T_PAD = 128


def _body(t_hbm, ls_hbm, lsm_hbm, o0_hbm, o1_hbm, o2_hbm, o3_hbm,
          idx_v, ls_v, lsm_v, ta_v, tr_v, o0_v, o1_v, o2_v, o3_v):
    wid = lax.axis_index("s") * NC + lax.axis_index("c")
    base = wid * BPW

    pltpu.sync_copy(t_hbm.at[pl.ds(base, BPW)], idx_v)
    pltpu.sync_copy(ls_hbm, ls_v)
    pltpu.sync_copy(lsm_hbm, lsm_v)

    for j in range(T_PAD // L):
        sl = pl.ds(j * L, L)
        x = ls_v[sl]
        xm = lsm_v[sl]
        e = jnp.exp(x)
        em = jnp.exp(xm)
        ta_v[sl] = e / (1.0 + e)
        tr_v[sl] = jnp.exp(x - xm) * ((1.0 + em) / (1.0 + e))

    for i in range(BPW // L):
        sl = pl.ds(i * L, L)
        tv = idx_v[sl]
        a = plsc.load_gather(ta_v, [tv])
        r = plsc.load_gather(tr_v, [tv])
        o0_v[sl] = a
        o1_v[sl] = 1.0 - r
        o2_v[sl] = 1.0 - a
        o3_v[sl] = r

    pltpu.sync_copy(o0_v, o0_hbm.at[pl.ds(base, BPW)])
    pltpu.sync_copy(o1_v, o1_hbm.at[pl.ds(base, BPW)])
    pltpu.sync_copy(o2_v, o2_hbm.at[pl.ds(base, BPW)])
    pltpu.sync_copy(o3_v, o3_hbm.at[pl.ds(base, BPW)])


_sched_kernel = functools.partial(
    pl.kernel,
    out_type=tuple(jax.ShapeDtypeStruct((B,), jnp.float32) for _ in range(4)),
    mesh=plsc.VectorSubcoreMesh(
        core_axis_name="c", subcore_axis_name="s",
        num_cores=NC, num_subcores=NS),
    compiler_params=pltpu.CompilerParams(needs_layout_passes=False),
    scratch_types=[
        pltpu.VMEM((BPW,), jnp.int32),
        pltpu.VMEM((T_PAD,), jnp.float32),
        pltpu.VMEM((T_PAD,), jnp.float32),
        pltpu.VMEM((T_PAD,), jnp.float32),
        pltpu.VMEM((T_PAD,), jnp.float32),
        pltpu.VMEM((BPW,), jnp.float32),
        pltpu.VMEM((BPW,), jnp.float32),
        pltpu.VMEM((BPW,), jnp.float32),
        pltpu.VMEM((BPW,), jnp.float32),
    ],
)(_body)


@jax.jit
def kernel(t, log_snr):
    n = log_snr.shape[0]
    ls = jnp.pad(log_snr, (0, T_PAD - n))
    lsm = jnp.concatenate([ls[:1], ls[:-1]])
    return _sched_kernel(t.astype(jnp.int32), ls, lsm)


# speedup vs baseline: 11.7679x; 1.0602x over previous
import functools

import jax
import jax.numpy as jnp
from jax import lax
from jax.experimental import pallas as pl
from jax.experimental.pallas import tpu as pltpu
from jax.experimental.pallas import tpu_sc as plsc

NC = 2
NS = 16
NW = NC * NS
L = 16

B = 16384
BPW = B // NW
N_STEPS = 100
T_PAD = 112


def _body(t_hbm, ls_hbm, o0_hbm, o1_hbm, o2_hbm, o3_hbm,
          idx_v, ls_v, ta_v, tr_v, o0_v, o1_v, o2_v, o3_v,
          sem_idx, sem_ls, sem_out):
    wid = lax.axis_index("s") * NC + lax.axis_index("c")
    base = wid * BPW

    cp_idx = pltpu.async_copy(t_hbm.at[pl.ds(base, BPW)], idx_v, sem_idx)
    cp_ls = pltpu.async_copy(ls_hbm, ls_v.at[pl.ds(0, N_STEPS)], sem_ls)
    cp_ls.wait()

    for j in range(T_PAD // L):
        ids = lax.iota(jnp.int32, L) + (j * L)
        x = ls_v[pl.ds(j * L, L)]
        xm = plsc.load_gather(ls_v, [jnp.maximum(ids - 1, 0)])
        e = jnp.exp(x)
        em = jnp.exp(xm)
        ta_v[pl.ds(j * L, L)] = e / (1.0 + e)
        tr_v[pl.ds(j * L, L)] = jnp.exp(x - xm) * ((1.0 + em) / (1.0 + e))

    cp_idx.wait()

    for i in range(BPW // L):
        sl = pl.ds(i * L, L)
        tv = idx_v[sl]
        a = plsc.load_gather(ta_v, [tv])
        r = plsc.load_gather(tr_v, [tv])
        o0_v[sl] = a
        o1_v[sl] = 1.0 - r
        o2_v[sl] = 1.0 - a
        o3_v[sl] = r

    cp0 = pltpu.async_copy(o0_v, o0_hbm.at[pl.ds(base, BPW)], sem_out)
    cp1 = pltpu.async_copy(o1_v, o1_hbm.at[pl.ds(base, BPW)], sem_out)
    cp2 = pltpu.async_copy(o2_v, o2_hbm.at[pl.ds(base, BPW)], sem_out)
    cp3 = pltpu.async_copy(o3_v, o3_hbm.at[pl.ds(base, BPW)], sem_out)
    cp0.wait()
    cp1.wait()
    cp2.wait()
    cp3.wait()


_sched_kernel = functools.partial(
    pl.kernel,
    out_type=tuple(jax.ShapeDtypeStruct((B,), jnp.float32) for _ in range(4)),
    mesh=plsc.VectorSubcoreMesh(
        core_axis_name="c", subcore_axis_name="s",
        num_cores=NC, num_subcores=NS),
    compiler_params=pltpu.CompilerParams(needs_layout_passes=False),
    scratch_types=[
        pltpu.VMEM((BPW,), jnp.int32),
        pltpu.VMEM((T_PAD,), jnp.float32),
        pltpu.VMEM((T_PAD,), jnp.float32),
        pltpu.VMEM((T_PAD,), jnp.float32),
        pltpu.VMEM((BPW,), jnp.float32),
        pltpu.VMEM((BPW,), jnp.float32),
        pltpu.VMEM((BPW,), jnp.float32),
        pltpu.VMEM((BPW,), jnp.float32),
        pltpu.SemaphoreType.DMA,
        pltpu.SemaphoreType.DMA,
        pltpu.SemaphoreType.DMA,
    ],
)(_body)


@jax.jit
def kernel(t, log_snr):
    return _sched_kernel(t.astype(jnp.int32), log_snr)
